# Initial kernel scaffold; baseline (speedup 1.0000x reference)
import jax
import jax.numpy as jnp
from jax import lax
from jax.experimental import pallas as pl
from jax.experimental.pallas import tpu as pltpu
from jax.experimental.pallas import tpu_sc as plsc

_NWORK = 32
_GB = 512
_MSGW = 160


def _split_gvp(p, si_parts):
    out = {}
    ws = p['ws_w']
    off = 0
    for name, w in si_parts:
        out['ws_' + name] = ws[off:off + w]
        off += w
    h = p['wh'].shape[1]
    out['ws_vn'] = ws[off:off + h]
    out['b'] = p['ws_b'][None, :]
    out['wv'] = p['wv']
    out['wsv'] = p['wsv_w']
    out['bsv'] = p['wsv_b'][None, :]
    return out


def _prep_weights(params):
    W = {}
    p1 = params['m1']
    ns = params['m2']['ws_w'].shape[1]
    se = p1['ws_w'].shape[0] - 2 * ns - p1['wh'].shape[1]
    nv = p1['wv'].shape[1]
    m1 = _split_gvp(p1, [('s', ns), ('e', se), ('d', ns)])
    W.update({'m1_' + k: v for k, v in m1.items()})
    W['m1_wh_s'] = p1['wh'][0:nv]
    W['m1_wh_e'] = p1['wh'][nv:nv + 1]
    W['m1_wh_d'] = p1['wh'][nv + 1:]
    for nm in ('m2', 'm3'):
        p = params[nm]
        g = _split_gvp(p, [('s', ns)])
        W.update({nm + '_' + k: v for k, v in g.items()})
        W[nm + '_wh'] = p['wh']
    for nm in ('f1', 'f2'):
        p = params[nm]
        si = p['ws_w'].shape[0] - p['wh'].shape[1]
        g = _split_gvp(p, [('s', si)])
        W.update({nm + '_' + k: v for k, v in g.items()})
        W[nm + '_wh'] = p['wh']
    for nm in ('ln0_g', 'ln0_b', 'ln1_g', 'ln1_b'):
        W[nm] = params[nm][None, :]
    return W


_MSG_W_NAMES = (
    'm1_wh_s', 'm1_wh_e', 'm1_wh_d', 'm1_ws_s', 'm1_ws_e', 'm1_ws_d',
    'm1_ws_vn', 'm1_b', 'm1_wv', 'm1_wsv', 'm1_bsv',
    'm2_wh', 'm2_ws_s', 'm2_ws_vn', 'm2_b', 'm2_wv', 'm2_wsv', 'm2_bsv',
    'm3_wh', 'm3_ws_s', 'm3_ws_vn', 'm3_b', 'm3_wv', 'm3_wsv', 'm3_bsv',
)

_NODE_W_NAMES = (
    'ln0_g', 'ln0_b',
    'f1_wh', 'f1_ws_s', 'f1_ws_vn', 'f1_b', 'f1_wv', 'f1_wsv', 'f1_bsv',
    'f2_wh', 'f2_ws_s', 'f2_ws_vn', 'f2_b', 'f2_wv', 'f2_wsv', 'f2_bsv',
    'ln1_g', 'ln1_b',
)


def _dot(a, b):
    return jax.lax.dot_general(a, b, (((1,), (0,)), ((), ())),
                               preferred_element_type=jnp.float32)


def _vnorm3(vh):
    return jnp.sqrt(jnp.clip(vh[0] * vh[0] + vh[1] * vh[1] + vh[2] * vh[2],
                             1e-8, None))


def _gvp_plain(s, v, W, pre, act):
    vh = [_dot(x, W[pre + '_wh']) for x in v]
    vn = _vnorm3(vh)
    so = _dot(s, W[pre + '_ws_s']) + _dot(vn, W[pre + '_ws_vn']) + W[pre + '_b']
    vo = [_dot(h, W[pre + '_wv']) for h in vh]
    g = jax.nn.sigmoid(so) if act else so
    gate = jax.nn.sigmoid(_dot(g, W[pre + '_wsv']) + W[pre + '_bsv'])
    vo = [o * gate for o in vo]
    if act:
        so = jax.nn.relu(so)
    return so, vo


def _message_math(ss, es, sd, vs, ev, vd, W):
    nv = W['m1_wh_s'].shape[0]
    vh = []
    for ci in range(3):
        vh.append(_dot(vs[:, nv * ci:nv * ci + nv], W['m1_wh_s'])
                  + _dot(vd[:, nv * ci:nv * ci + nv], W['m1_wh_d'])
                  + ev[:, ci:ci + 1] * W['m1_wh_e'])
    vn = _vnorm3(vh)
    s1 = (_dot(ss, W['m1_ws_s']) + _dot(es, W['m1_ws_e'])
          + _dot(sd, W['m1_ws_d']) + _dot(vn, W['m1_ws_vn']) + W['m1_b'])
    vo = [_dot(h, W['m1_wv']) for h in vh]
    gate = jax.nn.sigmoid(_dot(jax.nn.sigmoid(s1), W['m1_wsv']) + W['m1_bsv'])
    vo = [o * gate for o in vo]
    s1 = jax.nn.relu(s1)
    s2, vo = _gvp_plain(s1, vo, W, 'm2', act=True)
    s3, vo = _gvp_plain(s2, vo, W, 'm3', act=False)
    return s3, vo


def _gvp_layernorm(s, v, g, b):
    mu = jnp.mean(s, axis=1, keepdims=True)
    var = jnp.mean((s - mu) * (s - mu), axis=1, keepdims=True)
    s = (s - mu) * jax.lax.rsqrt(var + 1e-5) * g + b
    q = jnp.clip(v[0] * v[0] + v[1] * v[1] + v[2] * v[2], 1e-8, None)
    vn = jnp.sqrt(jnp.mean(q, axis=1, keepdims=True))
    inv = 1.0 / vn
    return s, [x * inv for x in v]


def _node_math(xs, xv, agg, W):
    ns = xs.shape[1]
    nv = xv.shape[1] // 3
    cnt = jnp.maximum(agg[:, ns + 3 * nv:ns + 3 * nv + 1], 1.0)
    inv = 1.0 / cnt
    s0 = xs + agg[:, 0:ns] * inv
    v0 = [xv[:, nv * c:nv * c + nv] + agg[:, ns + nv * c:ns + nv * c + nv] * inv
          for c in range(3)]
    s_ln, v_ln = _gvp_layernorm(s0, v0, W['ln0_g'], W['ln0_b'])
    ds_, dv = _gvp_plain(s_ln, v_ln, W, 'f1', act=True)
    ds_, dv = _gvp_plain(ds_, dv, W, 'f2', act=False)
    s_out, v_out = _gvp_layernorm(s_ln + ds_,
                                  [a + bb for a, bb in zip(v_ln, dv)],
                                  W['ln1_g'], W['ln1_b'])
    return s_out, jnp.concatenate(v_out, axis=1)


def _sc_gather(xs, xv, src, dst):
    E = src.shape[0]
    ns = xs.shape[1]
    nv3 = xv.shape[1]
    epw = E // _NWORK
    ngb = epw // _GB
    tail0 = epw - _GB
    nbat = ngb + (1 if epw % _GB else 0)
    gk = _GB // 128
    mesh = plsc.VectorSubcoreMesh(core_axis_name="c", subcore_axis_name="s")
    out_type = (jax.ShapeDtypeStruct((E, ns), jnp.float32),
                jax.ShapeDtypeStruct((E, ns), jnp.float32),
                jax.ShapeDtypeStruct((E, nv3), jnp.float32),
                jax.ShapeDtypeStruct((E, nv3), jnp.float32))

    def body(xs_hbm, xv_hbm, src_hbm, dst_hbm,
             gss, gsd, gvs, gvd, idx1, bs, bv, sem):
        c = lax.axis_index("c")
        s = lax.axis_index("s")
        base = (s * 2 + c) * epw

        def one_side(idx_hbm, outs, outv, e0):
            pltpu.sync_copy(idx_hbm.at[pl.ds(e0, _GB)], idx1)
            cps = []
            for j in range(gk):
                sl = pl.ds(128 * j, 128)
                cps.append(pltpu.async_copy(
                    xs_hbm.at[idx1.at[sl]], bs.at[sl], sem))
                cps.append(pltpu.async_copy(
                    xv_hbm.at[idx1.at[sl]], bv.at[sl], sem))
            for cp in cps:
                cp.wait()
            pltpu.sync_copy(bs, outs.at[pl.ds(e0, _GB)])
            pltpu.sync_copy(bv, outv.at[pl.ds(e0, _GB)])

        def step(k, carry):
            e0 = base + jnp.where(k < ngb, k * _GB, tail0)
            one_side(src_hbm, gss, gvs, e0)
            one_side(dst_hbm, gsd, gvd, e0)
            return carry

        lax.fori_loop(0, nbat, step, 0)

    fn = pl.kernel(body, out_type=out_type, mesh=mesh,
                   scratch_types=(pltpu.VMEM((_GB,), jnp.int32),
                                  pltpu.VMEM((_GB, ns), jnp.float32),
                                  pltpu.VMEM((_GB, nv3), jnp.float32),
                                  pltpu.SemaphoreType.DMA))
    return fn(xs, xv, src, dst)


def _tc_message(gss, gsd, gvs, gvd, es, ev, W):
    E = gss.shape[0]
    be = 1600
    grid = (E // be,)
    weights = [W[n] for n in _MSG_W_NAMES]

    def body(gss_r, gsd_r, gvs_r, gvd_r, es_r, ev_r, *refs):
        wrefs = refs[:-1]
        out_r = refs[-1]
        w = {n: r[...] for n, r in zip(_MSG_W_NAMES, wrefs)}
        s3, vo = _message_math(gss_r[...], es_r[...], gsd_r[...],
                               gvs_r[...], ev_r[...], gvd_r[...], w)
        bsz = s3.shape[0]
        ones = jnp.ones((bsz, 1), jnp.float32)
        pad = jnp.zeros((bsz, _MSGW - s3.shape[1] - 3 * vo[0].shape[1] - 1),
                        jnp.float32)
        out_r[...] = jnp.concatenate([s3] + vo + [ones, pad], axis=1)

    data_specs = [
        pl.BlockSpec((be, gss.shape[1]), lambda i: (i, 0)),
        pl.BlockSpec((be, gsd.shape[1]), lambda i: (i, 0)),
        pl.BlockSpec((be, gvs.shape[1]), lambda i: (i, 0)),
        pl.BlockSpec((be, gvd.shape[1]), lambda i: (i, 0)),
        pl.BlockSpec((be, es.shape[1]), lambda i: (i, 0)),
        pl.BlockSpec((be, ev.shape[1]), lambda i: (i, 0)),
    ]
    w_specs = [pl.BlockSpec(w.shape, lambda i: (0, 0)) for w in weights]
    return pl.pallas_call(
        body,
        grid=grid,
        in_specs=data_specs + w_specs,
        out_specs=pl.BlockSpec((be, _MSGW), lambda i: (i, 0)),
        out_shape=jax.ShapeDtypeStruct((E, _MSGW), jnp.float32),
        compiler_params=pltpu.CompilerParams(
            dimension_semantics=("arbitrary",)),
    )(gss, gsd, gvs, gvd, es, ev, *weights)


def _sc_scatter(msg, dst2d, zrows):
    E = msg.shape[0]
    N = zrows.shape[0] * 16
    nchunks = E // 128
    cpt = -(-nchunks // 16)
    rpt = N // 16
    mesh = plsc.VectorSubcoreMesh(core_axis_name="c", subcore_axis_name="s")
    out_type = jax.ShapeDtypeStruct((N, _MSGW), jnp.float32)

    def body(msg_hbm, dst_hbm, z_hbm, out_hbm, idx2, mbuf, acc):
        c = lax.axis_index("c")
        t = lax.axis_index("s")
        for q in range(2):
            col0 = 80 * c + 40 * q
            pltpu.sync_copy(z_hbm, acc.at[pl.ds(t * rpt, rpt)])
            plsc.subcore_barrier()

            def step(k, carry):
                cid = t + 16 * k

                @pl.when(cid < nchunks)
                def _():
                    pltpu.sync_copy(dst_hbm.at[pl.ds(cid, 1)], idx2)
                    pltpu.sync_copy(
                        msg_hbm.at[pl.ds(cid * 128, 128), pl.ds(col0, 40)],
                        mbuf)
                    pltpu.sync_copy(mbuf, acc.at[idx2.at[0]], add=True)

                return carry

            lax.fori_loop(0, cpt, step, 0)
            plsc.subcore_barrier()
            pltpu.sync_copy(acc.at[pl.ds(t * rpt, rpt)],
                            out_hbm.at[pl.ds(t * rpt, rpt), pl.ds(col0, 40)])
            plsc.subcore_barrier()

    fn = pl.kernel(body, out_type=out_type, mesh=mesh,
                   scratch_types=(pltpu.VMEM((1, 128), jnp.int32),
                                  pltpu.VMEM((128, 40), jnp.float32),
                                  pltpu.VMEM_SHARED((N, 40), jnp.float32)))
    return fn(msg, dst2d, zrows)


def _tc_node(xs, xv, agg, W):
    N = xs.shape[0]
    bn = 2000
    grid = (N // bn,)
    weights = [W[n] for n in _NODE_W_NAMES]

    def body(xs_r, xv_r, agg_r, *refs):
        wrefs = refs[:-2]
        outs_r, outv_r = refs[-2], refs[-1]
        w = {n: r[...] for n, r in zip(_NODE_W_NAMES, wrefs)}
        s_out, v_out = _node_math(xs_r[...], xv_r[...], agg_r[...], w)
        outs_r[...] = s_out
        outv_r[...] = v_out

    data_specs = [
        pl.BlockSpec((bn, xs.shape[1]), lambda i: (i, 0)),
        pl.BlockSpec((bn, xv.shape[1]), lambda i: (i, 0)),
        pl.BlockSpec((bn, _MSGW), lambda i: (i, 0)),
    ]
    w_specs = [pl.BlockSpec(w.shape, lambda i: (0, 0)) for w in weights]
    return pl.pallas_call(
        body,
        grid=grid,
        in_specs=data_specs + w_specs,
        out_specs=(pl.BlockSpec((bn, xs.shape[1]), lambda i: (i, 0)),
                   pl.BlockSpec((bn, xv.shape[1]), lambda i: (i, 0))),
        out_shape=(jax.ShapeDtypeStruct((N, xs.shape[1]), jnp.float32),
                   jax.ShapeDtypeStruct((N, xv.shape[1]), jnp.float32)),
        compiler_params=pltpu.CompilerParams(
            dimension_semantics=("arbitrary",)),
    )(xs, xv, agg, *weights)


def kernel(x_s, x_v, edge_index, edge_attr_s, edge_attr_v, params):
    N, ns = x_s.shape
    nv = x_v.shape[1]
    E = edge_index.shape[1]
    src, dst = edge_index[0], edge_index[1]
    xv_p = jnp.swapaxes(x_v, 1, 2).reshape(N, 3 * nv)
    ev = edge_attr_v.reshape(E, 3)
    W = _prep_weights(params)

    gss, gsd, gvs, gvd = _sc_gather(x_s, xv_p, src, dst)
    msg = _tc_message(gss, gsd, gvs, gvd, edge_attr_s, ev, W)
    dst2d = dst.reshape(E // 128, 128)
    zrows = jnp.zeros((N // 16, 40), jnp.float32)
    agg = _sc_scatter(msg, dst2d, zrows)
    out_s, out_vp = _tc_node(x_s, xv_p, agg, W)
    out_v = out_vp.reshape(N, 3, nv).transpose(0, 2, 1)
    return out_s, out_v


# speedup vs baseline: 11.2417x; 11.2417x over previous
import jax
import jax.numpy as jnp
from jax import lax
from jax.experimental import pallas as pl
from jax.experimental.pallas import tpu as pltpu
from jax.experimental.pallas import tpu_sc as plsc

_NWORK = 32
_GB = 512
_MSGW = 160


def _split_gvp(p, si_parts):
    out = {}
    ws = p['ws_w']
    off = 0
    for name, w in si_parts:
        out['ws_' + name] = ws[off:off + w]
        off += w
    h = p['wh'].shape[1]
    out['ws_vn'] = ws[off:off + h]
    out['b'] = p['ws_b'][None, :]
    out['wv'] = p['wv']
    out['wsv'] = p['wsv_w']
    out['bsv'] = p['wsv_b'][None, :]
    return out


def _prep_weights(params):
    W = {}
    p1 = params['m1']
    ns = params['m2']['ws_w'].shape[1]
    se = p1['ws_w'].shape[0] - 2 * ns - p1['wh'].shape[1]
    nv = p1['wv'].shape[1]
    m1 = _split_gvp(p1, [('s', ns), ('e', se), ('d', ns)])
    W.update({'m1_' + k: v for k, v in m1.items()})
    nsp = -(-ns // 16) * 16
    for k in ('m1_ws_s', 'm1_ws_d'):
        W[k] = jnp.concatenate(
            [W[k], jnp.zeros((nsp - ns, W[k].shape[1]), W[k].dtype)], axis=0)
    W['m1_wh_s'] = p1['wh'][0:nv]
    W['m1_wh_e'] = p1['wh'][nv:nv + 1]
    W['m1_wh_d'] = p1['wh'][nv + 1:]
    for nm in ('m2', 'm3'):
        p = params[nm]
        g = _split_gvp(p, [('s', ns)])
        W.update({nm + '_' + k: v for k, v in g.items()})
        W[nm + '_wh'] = p['wh']
    for nm in ('f1', 'f2'):
        p = params[nm]
        si = p['ws_w'].shape[0] - p['wh'].shape[1]
        g = _split_gvp(p, [('s', si)])
        W.update({nm + '_' + k: v for k, v in g.items()})
        W[nm + '_wh'] = p['wh']
    for nm in ('ln0_g', 'ln0_b', 'ln1_g', 'ln1_b'):
        W[nm] = params[nm][None, :]
    return W


_MSG_W_NAMES = (
    'm1_wh_s', 'm1_wh_e', 'm1_wh_d', 'm1_ws_s', 'm1_ws_e', 'm1_ws_d',
    'm1_ws_vn', 'm1_b', 'm1_wv', 'm1_wsv', 'm1_bsv',
    'm2_wh', 'm2_ws_s', 'm2_ws_vn', 'm2_b', 'm2_wv', 'm2_wsv', 'm2_bsv',
    'm3_wh', 'm3_ws_s', 'm3_ws_vn', 'm3_b', 'm3_wv', 'm3_wsv', 'm3_bsv',
)

_NODE_W_NAMES = (
    'ln0_g', 'ln0_b',
    'f1_wh', 'f1_ws_s', 'f1_ws_vn', 'f1_b', 'f1_wv', 'f1_wsv', 'f1_bsv',
    'f2_wh', 'f2_ws_s', 'f2_ws_vn', 'f2_b', 'f2_wv', 'f2_wsv', 'f2_bsv',
    'ln1_g', 'ln1_b',
)


def _dot(a, b):
    return jax.lax.dot_general(a, b, (((1,), (0,)), ((), ())),
                               preferred_element_type=jnp.float32)


def _vnorm3(vh):
    return jnp.sqrt(jnp.clip(vh[0] * vh[0] + vh[1] * vh[1] + vh[2] * vh[2],
                             1e-8, None))


def _gvp_plain(s, v, W, pre, act):
    vh = [_dot(x, W[pre + '_wh']) for x in v]
    vn = _vnorm3(vh)
    so = _dot(s, W[pre + '_ws_s']) + _dot(vn, W[pre + '_ws_vn']) + W[pre + '_b']
    vo = [_dot(h, W[pre + '_wv']) for h in vh]
    g = jax.nn.sigmoid(so) if act else so
    gate = jax.nn.sigmoid(_dot(g, W[pre + '_wsv']) + W[pre + '_bsv'])
    vo = [o * gate for o in vo]
    if act:
        so = jax.nn.relu(so)
    return so, vo


def _message_math(ss, es, sd, vs, ev, vd, W):
    nv = W['m1_wh_s'].shape[0]
    vh = []
    for ci in range(3):
        vh.append(_dot(vs[:, nv * ci:nv * ci + nv], W['m1_wh_s'])
                  + _dot(vd[:, nv * ci:nv * ci + nv], W['m1_wh_d'])
                  + ev[:, ci:ci + 1] * W['m1_wh_e'])
    vn = _vnorm3(vh)
    s1 = (_dot(ss, W['m1_ws_s']) + _dot(es, W['m1_ws_e'])
          + _dot(sd, W['m1_ws_d']) + _dot(vn, W['m1_ws_vn']) + W['m1_b'])
    vo = [_dot(h, W['m1_wv']) for h in vh]
    gate = jax.nn.sigmoid(_dot(jax.nn.sigmoid(s1), W['m1_wsv']) + W['m1_bsv'])
    vo = [o * gate for o in vo]
    s1 = jax.nn.relu(s1)
    s2, vo = _gvp_plain(s1, vo, W, 'm2', act=True)
    s3, vo = _gvp_plain(s2, vo, W, 'm3', act=False)
    return s3, vo


def _gvp_layernorm(s, v, g, b):
    mu = jnp.mean(s, axis=1, keepdims=True)
    var = jnp.mean((s - mu) * (s - mu), axis=1, keepdims=True)
    s = (s - mu) * jax.lax.rsqrt(var + 1e-5) * g + b
    q = jnp.clip(v[0] * v[0] + v[1] * v[1] + v[2] * v[2], 1e-8, None)
    vn = jnp.sqrt(jnp.mean(q, axis=1, keepdims=True))
    inv = 1.0 / vn
    return s, [x * inv for x in v]


def _node_math(xs, xv, agg, W):
    ns = xs.shape[1]
    nv = xv.shape[1] // 3
    cnt = jnp.maximum(agg[:, ns + 3 * nv:ns + 3 * nv + 1], 1.0)
    inv = 1.0 / cnt
    s0 = xs + agg[:, 0:ns] * inv
    v0 = [xv[:, nv * c:nv * c + nv] + agg[:, ns + nv * c:ns + nv * c + nv] * inv
          for c in range(3)]
    s_ln, v_ln = _gvp_layernorm(s0, v0, W['ln0_g'], W['ln0_b'])
    ds_, dv = _gvp_plain(s_ln, v_ln, W, 'f1', act=True)
    ds_, dv = _gvp_plain(ds_, dv, W, 'f2', act=False)
    s_out, v_out = _gvp_layernorm(s_ln + ds_,
                                  [a + bb for a, bb in zip(v_ln, dv)],
                                  W['ln1_g'], W['ln1_b'])
    return s_out, jnp.concatenate(v_out, axis=1)


def _sc_gather(xs, xv, src, dst):
    E = src.shape[0]
    ns = xs.shape[1]
    nv3 = xv.shape[1]
    epw = E // _NWORK
    ngb = epw // _GB
    tail0 = epw - _GB
    nbat = ngb + (1 if epw % _GB else 0)
    gk = _GB // 128
    mesh = plsc.VectorSubcoreMesh(core_axis_name="c", subcore_axis_name="s")
    out_type = (jax.ShapeDtypeStruct((E, ns), jnp.float32),
                jax.ShapeDtypeStruct((E, ns), jnp.float32),
                jax.ShapeDtypeStruct((E, nv3), jnp.float32),
                jax.ShapeDtypeStruct((E, nv3), jnp.float32))

    def body(xs_hbm, xv_hbm, src_hbm, dst_hbm,
             gss, gsd, gvs, gvd, idx1, bs, bv, sem):
        c = lax.axis_index("c")
        s = lax.axis_index("s")
        base = (s * 2 + c) * epw

        def one_side(idx_hbm, outs, outv, e0):
            pltpu.sync_copy(idx_hbm.at[pl.ds(e0, _GB)], idx1)
            cps = []
            for j in range(gk):
                sl = pl.ds(128 * j, 128)
                cps.append(pltpu.async_copy(
                    xs_hbm.at[idx1.at[sl]], bs.at[sl], sem))
                cps.append(pltpu.async_copy(
                    xv_hbm.at[idx1.at[sl]], bv.at[sl], sem))
            for cp in cps:
                cp.wait()
            pltpu.sync_copy(bs, outs.at[pl.ds(e0, _GB)])
            pltpu.sync_copy(bv, outv.at[pl.ds(e0, _GB)])

        def step(k, carry):
            e0 = base + jnp.where(k < ngb, k * _GB, tail0)
            one_side(src_hbm, gss, gvs, e0)
            one_side(dst_hbm, gsd, gvd, e0)
            return carry

        lax.fori_loop(0, nbat, step, 0)

    fn = pl.kernel(body, out_type=out_type, mesh=mesh,
                   compiler_params=pltpu.CompilerParams(
                       use_tc_tiling_on_sc=False),
                   scratch_types=(pltpu.VMEM((_GB,), jnp.int32),
                                  pltpu.VMEM((_GB, ns), jnp.float32),
                                  pltpu.VMEM((_GB, nv3), jnp.float32),
                                  pltpu.SemaphoreType.DMA))
    return fn(xs, xv, src, dst)


def _tc_message(gss, gsd, gvs, gvd, es, ev, W):
    E = gss.shape[0]
    be = 1600
    grid = (E // be,)
    weights = [W[n] for n in _MSG_W_NAMES]

    def body(gss_r, gsd_r, gvs_r, gvd_r, es_r, ev_r, *refs):
        wrefs = refs[:-1]
        out_r = refs[-1]
        w = {n: r[...] for n, r in zip(_MSG_W_NAMES, wrefs)}
        s3, vo = _message_math(gss_r[...], es_r[...], gsd_r[...],
                               gvs_r[...], ev_r[...], gvd_r[...], w)
        bsz = s3.shape[0]
        ones = jnp.ones((bsz, 1), jnp.float32)
        pad = jnp.zeros((bsz, _MSGW - s3.shape[1] - 3 * vo[0].shape[1] - 1),
                        jnp.float32)
        out_r[...] = jnp.concatenate([s3] + vo + [ones, pad], axis=1)

    data_specs = [
        pl.BlockSpec((be, gss.shape[1]), lambda i: (i, 0)),
        pl.BlockSpec((be, gsd.shape[1]), lambda i: (i, 0)),
        pl.BlockSpec((be, gvs.shape[1]), lambda i: (i, 0)),
        pl.BlockSpec((be, gvd.shape[1]), lambda i: (i, 0)),
        pl.BlockSpec((be, es.shape[1]), lambda i: (i, 0)),
        pl.BlockSpec((be, ev.shape[1]), lambda i: (i, 0)),
    ]
    w_specs = [pl.BlockSpec(w.shape, lambda i: (0, 0)) for w in weights]
    return pl.pallas_call(
        body,
        grid=grid,
        in_specs=data_specs + w_specs,
        out_specs=pl.BlockSpec((be, _MSGW), lambda i: (i, 0)),
        out_shape=jax.ShapeDtypeStruct((E, _MSGW), jnp.float32),
        compiler_params=pltpu.CompilerParams(
            dimension_semantics=("arbitrary",)),
    )(gss, gsd, gvs, gvd, es, ev, *weights)


def _sc_scatter(msg, dst2d, zrows):
    E = msg.shape[0]
    N = zrows.shape[0] * 16
    nchunks = E // 128
    cpt = -(-nchunks // 16)
    rpt = N // 16
    mesh = plsc.VectorSubcoreMesh(core_axis_name="c", subcore_axis_name="s")
    out_type = jax.ShapeDtypeStruct((N, _MSGW), jnp.float32)

    def body(msg_hbm, dst_hbm, z_hbm, out_hbm, idx2, mbuf, acc):
        c = lax.axis_index("c")
        t = lax.axis_index("s")
        for q in range(2):
            col0 = 80 * c + 40 * q
            pltpu.sync_copy(z_hbm, acc.at[pl.ds(t * rpt, rpt)])
            plsc.subcore_barrier()

            def step(k, carry):
                cid = t + 16 * k

                @pl.when(cid < nchunks)
                def _():
                    pltpu.sync_copy(dst_hbm.at[pl.ds(cid, 1)], idx2)
                    pltpu.sync_copy(
                        msg_hbm.at[pl.ds(cid * 128, 128), pl.ds(col0, 40)],
                        mbuf)
                    pltpu.sync_copy(mbuf, acc.at[idx2.at[0]], add=True)

                return carry

            lax.fori_loop(0, cpt, step, 0)
            plsc.subcore_barrier()
            pltpu.sync_copy(acc.at[pl.ds(t * rpt, rpt)],
                            out_hbm.at[pl.ds(t * rpt, rpt), pl.ds(col0, 40)])
            plsc.subcore_barrier()

    fn = pl.kernel(body, out_type=out_type, mesh=mesh,
                   compiler_params=pltpu.CompilerParams(
                       use_tc_tiling_on_sc=False),
                   scratch_types=(pltpu.VMEM((1, 128), jnp.int32),
                                  pltpu.VMEM((128, 40), jnp.float32),
                                  pltpu.VMEM_SHARED((N, 40), jnp.float32)))
    return fn(msg, dst2d, zrows)


def _tc_node(xs, xv, agg, W):
    N = xs.shape[0]
    bn = 2000
    grid = (N // bn,)
    weights = [W[n] for n in _NODE_W_NAMES]

    def body(xs_r, xv_r, agg_r, *refs):
        wrefs = refs[:-2]
        outs_r, outv_r = refs[-2], refs[-1]
        w = {n: r[...] for n, r in zip(_NODE_W_NAMES, wrefs)}
        s_out, v_out = _node_math(xs_r[...], xv_r[...], agg_r[...], w)
        outs_r[...] = s_out
        outv_r[...] = v_out

    data_specs = [
        pl.BlockSpec((bn, xs.shape[1]), lambda i: (i, 0)),
        pl.BlockSpec((bn, xv.shape[1]), lambda i: (i, 0)),
        pl.BlockSpec((bn, _MSGW), lambda i: (i, 0)),
    ]
    w_specs = [pl.BlockSpec(w.shape, lambda i: (0, 0)) for w in weights]
    return pl.pallas_call(
        body,
        grid=grid,
        in_specs=data_specs + w_specs,
        out_specs=(pl.BlockSpec((bn, xs.shape[1]), lambda i: (i, 0)),
                   pl.BlockSpec((bn, xv.shape[1]), lambda i: (i, 0))),
        out_shape=(jax.ShapeDtypeStruct((N, xs.shape[1]), jnp.float32),
                   jax.ShapeDtypeStruct((N, xv.shape[1]), jnp.float32)),
        compiler_params=pltpu.CompilerParams(
            dimension_semantics=("arbitrary",)),
    )(xs, xv, agg, *weights)


def kernel(x_s, x_v, edge_index, edge_attr_s, edge_attr_v, params):
    N, ns = x_s.shape
    nv = x_v.shape[1]
    E = edge_index.shape[1]
    src, dst = edge_index[0], edge_index[1]
    xv_p = jnp.swapaxes(x_v, 1, 2).reshape(N, 3 * nv)
    ev = edge_attr_v.reshape(E, 3)
    W = _prep_weights(params)

    nsp = -(-ns // 16) * 16
    xs_pad = jnp.concatenate(
        [x_s, jnp.zeros((N, nsp - ns), x_s.dtype)], axis=1)
    gss, gsd, gvs, gvd = _sc_gather(xs_pad, xv_p, src, dst)
    msg = _tc_message(gss, gsd, gvs, gvd, edge_attr_s, ev, W)
    dst2d = dst.reshape(E // 128, 128)
    zrows = jnp.zeros((N // 16, 40), jnp.float32)
    agg = _sc_scatter(msg, dst2d, zrows)
    out_s, out_vp = _tc_node(x_s, xv_p, agg, W)
    out_v = out_vp.reshape(N, 3, nv).transpose(0, 2, 1)
    return out_s, out_v


# speedup vs baseline: 13.3854x; 1.1907x over previous
import jax
import jax.numpy as jnp
from jax import lax
from jax.experimental import pallas as pl
from jax.experimental.pallas import tpu as pltpu
from jax.experimental.pallas import tpu_sc as plsc

_NWORK = 32
_GB = 512
_SCW = 40
_SCR = 1


def _split_gvp(p, si_parts):
    out = {}
    ws = p['ws_w']
    off = 0
    for name, w in si_parts:
        out['ws_' + name] = ws[off:off + w]
        off += w
    h = p['wh'].shape[1]
    out['ws_vn'] = ws[off:off + h]
    out['b'] = p['ws_b'][None, :]
    out['wv'] = p['wv']
    out['wsv'] = p['wsv_w']
    out['bsv'] = p['wsv_b'][None, :]
    return out


def _prep_weights(params):
    W = {}
    p1 = params['m1']
    ns = params['m2']['ws_w'].shape[1]
    se = p1['ws_w'].shape[0] - 2 * ns - p1['wh'].shape[1]
    nv = p1['wv'].shape[1]
    m1 = _split_gvp(p1, [('s', ns), ('e', se), ('d', ns)])
    W.update({'m1_' + k: v for k, v in m1.items()})
    for k in ('m1_ws_s', 'm1_ws_d'):
        W[k] = jnp.concatenate(
            [W[k], jnp.zeros((128 - ns, W[k].shape[1]), W[k].dtype)], axis=0)
    W['m1_wh_s'] = p1['wh'][0:nv]
    W['m1_wh_e'] = p1['wh'][nv:nv + 1]
    W['m1_wh_d'] = p1['wh'][nv + 1:]
    for nm in ('m2', 'm3'):
        p = params[nm]
        g = _split_gvp(p, [('s', ns)])
        W.update({nm + '_' + k: v for k, v in g.items()})
        W[nm + '_wh'] = p['wh']
    for nm in ('f1', 'f2'):
        p = params[nm]
        si = p['ws_w'].shape[0] - p['wh'].shape[1]
        g = _split_gvp(p, [('s', si)])
        W.update({nm + '_' + k: v for k, v in g.items()})
        W[nm + '_wh'] = p['wh']
    for nm in ('ln0_g', 'ln0_b', 'ln1_g', 'ln1_b'):
        W[nm] = params[nm][None, :]
    return W


_MSG_W_NAMES = (
    'm1_wh_s', 'm1_wh_e', 'm1_wh_d', 'm1_ws_s', 'm1_ws_e', 'm1_ws_d',
    'm1_ws_vn', 'm1_b', 'm1_wv', 'm1_wsv', 'm1_bsv',
    'm2_wh', 'm2_ws_s', 'm2_ws_vn', 'm2_b', 'm2_wv', 'm2_wsv', 'm2_bsv',
    'm3_wh', 'm3_ws_s', 'm3_ws_vn', 'm3_b', 'm3_wv', 'm3_wsv', 'm3_bsv',
)

_NODE_W_NAMES = (
    'ln0_g', 'ln0_b',
    'f1_wh', 'f1_ws_s', 'f1_ws_vn', 'f1_b', 'f1_wv', 'f1_wsv', 'f1_bsv',
    'f2_wh', 'f2_ws_s', 'f2_ws_vn', 'f2_b', 'f2_wv', 'f2_wsv', 'f2_bsv',
    'ln1_g', 'ln1_b',
)


def _dot(a, b):
    return jax.lax.dot_general(a, b, (((1,), (0,)), ((), ())),
                               preferred_element_type=jnp.float32)


def _vnorm3(vh):
    return jnp.sqrt(jnp.clip(vh[0] * vh[0] + vh[1] * vh[1] + vh[2] * vh[2],
                             1e-8, None))


def _gvp_plain(s, v, W, pre, act):
    vh = [_dot(x, W[pre + '_wh']) for x in v]
    vn = _vnorm3(vh)
    so = _dot(s, W[pre + '_ws_s']) + _dot(vn, W[pre + '_ws_vn']) + W[pre + '_b']
    vo = [_dot(h, W[pre + '_wv']) for h in vh]
    g = jax.nn.sigmoid(so) if act else so
    gate = jax.nn.sigmoid(_dot(g, W[pre + '_wsv']) + W[pre + '_bsv'])
    vo = [o * gate for o in vo]
    if act:
        so = jax.nn.relu(so)
    return so, vo


def _message_math(ss, es, sd, vb, ev, W):
    nv = W['m1_wh_s'].shape[0]
    vh = []
    for ci in range(3):
        vh.append(_dot(vb[:, nv * ci:nv * ci + nv], W['m1_wh_s'])
                  + _dot(vb[:, 48 + nv * ci:48 + nv * ci + nv], W['m1_wh_d'])
                  + ev[:, ci:ci + 1] * W['m1_wh_e'])
    vn = _vnorm3(vh)
    s1 = (_dot(ss, W['m1_ws_s']) + _dot(es, W['m1_ws_e'])
          + _dot(sd, W['m1_ws_d']) + _dot(vn, W['m1_ws_vn']) + W['m1_b'])
    vo = [_dot(h, W['m1_wv']) for h in vh]
    gate = jax.nn.sigmoid(_dot(jax.nn.sigmoid(s1), W['m1_wsv']) + W['m1_bsv'])
    vo = [o * gate for o in vo]
    s1 = jax.nn.relu(s1)
    s2, vo = _gvp_plain(s1, vo, W, 'm2', act=True)
    s3, vo = _gvp_plain(s2, vo, W, 'm3', act=False)
    return s3, vo


def _gvp_layernorm(s, v, g, b):
    mu = jnp.mean(s, axis=1, keepdims=True)
    var = jnp.mean((s - mu) * (s - mu), axis=1, keepdims=True)
    s = (s - mu) * jax.lax.rsqrt(var + 1e-5) * g + b
    q = jnp.clip(v[0] * v[0] + v[1] * v[1] + v[2] * v[2], 1e-8, None)
    vn = jnp.sqrt(jnp.mean(q, axis=1, keepdims=True))
    inv = 1.0 / vn
    return s, [x * inv for x in v]


def _node_math(xs, xv, agg_a, agg_b, W):
    ns = xs.shape[1]
    nv = xv.shape[1] // 3
    cnt = jnp.maximum(agg_a[:, ns + nv:ns + nv + 1], 1.0)
    inv = 1.0 / cnt
    s0 = xs + agg_a[:, 0:ns] * inv
    aggv = [agg_a[:, ns:ns + nv], agg_b[:, 0:nv], agg_b[:, nv:2 * nv]]
    v0 = [xv[:, nv * c:nv * c + nv] + aggv[c] * inv for c in range(3)]
    s_ln, v_ln = _gvp_layernorm(s0, v0, W['ln0_g'], W['ln0_b'])
    ds_, dv = _gvp_plain(s_ln, v_ln, W, 'f1', act=True)
    ds_, dv = _gvp_plain(ds_, dv, W, 'f2', act=False)
    s_out, v_out = _gvp_layernorm(s_ln + ds_,
                                  [a + bb for a, bb in zip(v_ln, dv)],
                                  W['ln1_g'], W['ln1_b'])
    return s_out, jnp.concatenate(v_out, axis=1)


def _sc_gather(xs, xv, src, dst):
    E = src.shape[0]
    ns = xs.shape[1]
    nv3 = xv.shape[1]
    epw = E // _NWORK
    ngb = epw // _GB
    tail0 = epw - _GB
    nbat = ngb + (1 if epw % _GB else 0)
    gk = _GB // 128
    mesh = plsc.VectorSubcoreMesh(core_axis_name="c", subcore_axis_name="s")
    out_type = (jax.ShapeDtypeStruct((E, ns), jnp.float32),
                jax.ShapeDtypeStruct((E, ns), jnp.float32),
                jax.ShapeDtypeStruct((E, 128), jnp.float32))

    def body(xs_hbm, xv_hbm, src_hbm, dst_hbm,
             gss, gsd, gvb, idx1, bs, bvs, bvd, sem):
        c = lax.axis_index("c")
        s = lax.axis_index("s")
        base = (s * 2 + c) * epw

        def gather_side(idx_hbm, bv, e0):
            pltpu.sync_copy(idx_hbm.at[pl.ds(e0, _GB)], idx1)
            cps = []
            for j in range(gk):
                sl = pl.ds(128 * j, 128)
                cps.append(pltpu.async_copy(
                    xs_hbm.at[idx1.at[sl]], bs.at[sl], sem))
                cps.append(pltpu.async_copy(
                    xv_hbm.at[idx1.at[sl]], bv.at[sl], sem))
            for cp in cps:
                cp.wait()

        def step(k, carry):
            e0 = base + jnp.where(k < ngb, k * _GB, tail0)
            gather_side(src_hbm, bvs, e0)
            pltpu.sync_copy(bs, gss.at[pl.ds(e0, _GB)])
            gather_side(dst_hbm, bvd, e0)
            pltpu.sync_copy(bs, gsd.at[pl.ds(e0, _GB)])
            pltpu.sync_copy(bvs, gvb.at[pl.ds(e0, _GB), pl.ds(0, nv3)])
            pltpu.sync_copy(bvd, gvb.at[pl.ds(e0, _GB), pl.ds(nv3, nv3)])
            return carry

        lax.fori_loop(0, nbat, step, 0)

    fn = pl.kernel(body, out_type=out_type, mesh=mesh,
                   compiler_params=pltpu.CompilerParams(
                       use_tc_tiling_on_sc=False),
                   scratch_types=(pltpu.VMEM((_GB,), jnp.int32),
                                  pltpu.VMEM((_GB, ns), jnp.float32),
                                  pltpu.VMEM((_GB, nv3), jnp.float32),
                                  pltpu.VMEM((_GB, nv3), jnp.float32),
                                  pltpu.SemaphoreType.DMA))
    return fn(xs, xv, src, dst)


def _tc_message(gss, gsd, gvb, es, ev, W):
    E = gss.shape[0]
    be = 1600
    grid = (E // be,)
    weights = [W[n] for n in _MSG_W_NAMES]

    def body(gss_r, gsd_r, gvb_r, es_r, ev_r, *refs):
        wrefs = refs[:-2]
        outa_r, outb_r = refs[-2], refs[-1]
        w = {n: r[...] for n, r in zip(_MSG_W_NAMES, wrefs)}
        s3, vo = _message_math(gss_r[...], es_r[...], gsd_r[...],
                               gvb_r[...], ev_r[...], w)
        bsz = s3.shape[0]
        ones = jnp.ones((bsz, 1), jnp.float32)
        pada = jnp.zeros((bsz, 128 - 117), jnp.float32)
        padb = jnp.zeros((bsz, 128 - 32), jnp.float32)
        outa_r[...] = jnp.concatenate([s3, vo[0], ones, pada], axis=1)
        outb_r[...] = jnp.concatenate([vo[1], vo[2], padb], axis=1)

    data_specs = [
        pl.BlockSpec((be, gss.shape[1]), lambda i: (i, 0)),
        pl.BlockSpec((be, gsd.shape[1]), lambda i: (i, 0)),
        pl.BlockSpec((be, gvb.shape[1]), lambda i: (i, 0)),
        pl.BlockSpec((be, es.shape[1]), lambda i: (i, 0)),
        pl.BlockSpec((be, ev.shape[1]), lambda i: (i, 0)),
    ]
    w_specs = [pl.BlockSpec(w.shape, lambda i: (0, 0)) for w in weights]
    return pl.pallas_call(
        body,
        grid=grid,
        in_specs=data_specs + w_specs,
        out_specs=(pl.BlockSpec((be, 128), lambda i: (i, 0)),
                   pl.BlockSpec((be, 128), lambda i: (i, 0))),
        out_shape=(jax.ShapeDtypeStruct((E, 128), jnp.float32),
                   jax.ShapeDtypeStruct((E, 128), jnp.float32)),
        compiler_params=pltpu.CompilerParams(
            dimension_semantics=("arbitrary",)),
    )(gss, gsd, gvb, es, ev, *weights)


def _sc_scatter(msg_a, msg_b, dst2d, zrows):
    E = msg_a.shape[0]
    N = zrows.shape[0] * 16
    nrows = E // 128
    nchunk = nrows // _SCR
    cpt = -(-nchunk // 16)
    rpt = N // 16
    mesh = plsc.VectorSubcoreMesh(core_axis_name="c", subcore_axis_name="s")
    out_type = (jax.ShapeDtypeStruct((N, 128), jnp.float32),
                jax.ShapeDtypeStruct((N, 128), jnp.float32))
    passes = ((0, 0, 0), (0, 0, 40), (1, 0, 80), (1, 1, 0))

    def body(ma_hbm, mb_hbm, dst_hbm, z_hbm, outa_hbm, outb_hbm,
             idxs, mbuf, acc):
        c = lax.axis_index("c")
        t = lax.axis_index("s")
        row0 = t * rpt

        def one_pass(m_hbm, out_hbm, col0):
            pltpu.sync_copy(z_hbm.at[pl.ds(row0, rpt), pl.ds(0, _SCW)],
                            acc.at[pl.ds(row0, rpt)])
            plsc.subcore_barrier()

            def step(k, carry):
                cid = t + 16 * k

                @pl.when(cid < nchunk)
                def _():
                    pltpu.sync_copy(dst_hbm.at[pl.ds(_SCR * cid, _SCR)], idxs)
                    pltpu.sync_copy(
                        m_hbm.at[pl.ds(_SCR * 128 * cid, _SCR * 128),
                                 pl.ds(col0, _SCW)],
                        mbuf)
                    for j in range(_SCR):
                        pltpu.sync_copy(mbuf.at[pl.ds(128 * j, 128)],
                                        acc.at[idxs.at[j]], add=True)

                return carry

            lax.fori_loop(0, cpt, step, 0)
            plsc.subcore_barrier()
            pltpu.sync_copy(acc.at[pl.ds(row0, rpt)],
                            out_hbm.at[pl.ds(row0, rpt), pl.ds(col0, _SCW)])
            plsc.subcore_barrier()

        for cc, which, col0 in passes:
            @pl.when(c == cc)
            def _(which=which, col0=col0):
                one_pass(ma_hbm if which == 0 else mb_hbm,
                         outa_hbm if which == 0 else outb_hbm, col0)

    fn = pl.kernel(body, out_type=out_type, mesh=mesh,
                   compiler_params=pltpu.CompilerParams(
                       use_tc_tiling_on_sc=False),
                   scratch_types=(pltpu.VMEM((_SCR, 128), jnp.int32),
                                  pltpu.VMEM((_SCR * 128, _SCW), jnp.float32),
                                  pltpu.VMEM_SHARED((N, _SCW), jnp.float32)))
    return fn(msg_a, msg_b, dst2d, zrows)


def _tc_node(xs, xv, agg_a, agg_b, W):
    N = xs.shape[0]
    bn = 2000
    grid = (N // bn,)
    weights = [W[n] for n in _NODE_W_NAMES]

    def body(xs_r, xv_r, agga_r, aggb_r, *refs):
        wrefs = refs[:-2]
        outs_r, outv_r = refs[-2], refs[-1]
        w = {n: r[...] for n, r in zip(_NODE_W_NAMES, wrefs)}
        s_out, v_out = _node_math(xs_r[...], xv_r[...],
                                  agga_r[...], aggb_r[...], w)
        outs_r[...] = s_out
        outv_r[...] = v_out

    data_specs = [
        pl.BlockSpec((bn, xs.shape[1]), lambda i: (i, 0)),
        pl.BlockSpec((bn, xv.shape[1]), lambda i: (i, 0)),
        pl.BlockSpec((bn, 128), lambda i: (i, 0)),
        pl.BlockSpec((bn, 128), lambda i: (i, 0)),
    ]
    w_specs = [pl.BlockSpec(w.shape, lambda i: (0, 0)) for w in weights]
    return pl.pallas_call(
        body,
        grid=grid,
        in_specs=data_specs + w_specs,
        out_specs=(pl.BlockSpec((bn, xs.shape[1]), lambda i: (i, 0)),
                   pl.BlockSpec((bn, xv.shape[1]), lambda i: (i, 0))),
        out_shape=(jax.ShapeDtypeStruct((N, xs.shape[1]), jnp.float32),
                   jax.ShapeDtypeStruct((N, xv.shape[1]), jnp.float32)),
        compiler_params=pltpu.CompilerParams(
            dimension_semantics=("arbitrary",)),
    )(xs, xv, agg_a, agg_b, *weights)


def kernel(x_s, x_v, edge_index, edge_attr_s, edge_attr_v, params):
    N, ns = x_s.shape
    nv = x_v.shape[1]
    E = edge_index.shape[1]
    src, dst = edge_index[0], edge_index[1]
    xv_p = jnp.swapaxes(x_v, 1, 2).reshape(N, 3 * nv)
    ev = edge_attr_v.reshape(E, 3)
    W = _prep_weights(params)

    xs_pad = jnp.concatenate(
        [x_s, jnp.zeros((N, 128 - ns), x_s.dtype)], axis=1)
    gss, gsd, gvb = _sc_gather(xs_pad, xv_p, src, dst)
    msg_a, msg_b = _tc_message(gss, gsd, gvb, edge_attr_s, ev, W)
    dst2d = dst.reshape(E // 128, 128)
    zrows = jnp.zeros((N // 16, 128), jnp.float32)
    agg_a, agg_b = _sc_scatter(msg_a, msg_b, dst2d, zrows)
    out_s, out_vp = _tc_node(x_s, xv_p, agg_a, agg_b, W)
    out_v = out_vp.reshape(N, 3, nv).transpose(0, 2, 1)
    return out_s, out_v
